# Initial kernel scaffold; baseline (speedup 1.0000x reference)
import jax
import jax.numpy as jnp
from jax.experimental import pallas as pl


def kernel(idx, num, table):
    raise NotImplementedError("write your pallas kernel here")


# speedup vs baseline: 2.5973x; 2.5973x over previous
import functools

import jax
import jax.numpy as jnp
from jax import lax
from jax.experimental import pallas as pl
from jax.experimental.pallas import tpu as pltpu
from jax.experimental.pallas import tpu_sc as plsc

VOCAB = 1024
D = 128
LANES = 16
COLB = D // LANES


def _make_sc_kernel(n_total: int):
    info = plsc.get_sparse_core_info()
    nw = info.num_cores * info.num_subcores
    npw = n_total // nw
    chunk = 800
    nchunk = npw // chunk
    assert npw % chunk == 0

    mesh = plsc.VectorSubcoreMesh(core_axis_name="c", subcore_axis_name="s")

    @functools.partial(
        pl.kernel,
        mesh=mesh,
        out_type=jax.ShapeDtypeStruct((n_total, D), jnp.float32),
        scratch_types=[
            pltpu.VMEM((chunk,), jnp.int32),
            pltpu.VMEM((chunk,), jnp.float32),
            pltpu.VMEM((chunk, D), jnp.float32),
            pltpu.SemaphoreType.DMA,
        ],
    )
    def sc_embed(table_hbm, idx_hbm, num_hbm, out_hbm, idx_v, num_v, rows_v, sem):
        wid = lax.axis_index("s") * info.num_cores + lax.axis_index("c")
        wbase = wid * npw

        def chunk_body(c, _):
            base = wbase + c * chunk
            pltpu.sync_copy(idx_hbm.at[pl.ds(base, chunk)], idx_v)
            pltpu.sync_copy(num_hbm.at[pl.ds(base, chunk)], num_v)
            pltpu.async_copy(table_hbm.at[idx_v], rows_v, sem).wait()

            def group_body(g, _):
                num16 = num_v[pl.ds(g * LANES, LANES)]
                rbase = g * LANES
                for r in range(LANES):
                    s = num16[r]
                    for k in range(COLB):
                        blk = rows_v[rbase + r, pl.ds(k * LANES, LANES)]
                        rows_v[rbase + r, pl.ds(k * LANES, LANES)] = blk * s
                return 0

            lax.fori_loop(0, chunk // LANES, group_body, 0)
            pltpu.sync_copy(rows_v, out_hbm.at[pl.ds(base, chunk)])
            return 0

        lax.fori_loop(0, nchunk, chunk_body, 0)

    return sc_embed


def kernel(idx, num, table):
    b, t, dp = idx.shape
    n = b * t * dp
    idx_flat = idx.reshape(n).astype(jnp.int32)
    num_flat = num.reshape(n)
    out = _make_sc_kernel(n)(table, idx_flat, num_flat)
    return out.reshape(b, t, dp, D)
